# baseline (speedup 1.0000x reference)
import functools

import jax
import jax.numpy as jnp
from jax import lax
from jax.experimental import pallas as pl
from jax.experimental.pallas import tpu as pltpu
from jax.experimental.pallas import tpu_sc as plsc

E = 8
K = 2
D_IN = 1024
D_H = 2048
D_OUT = 1024
N = 4096

BLK = 256
NB = 40
CAP = NB * BLK
NW = 32
DH2 = D_H // 2

_GATE_BLK = 512


def _gate_body(x_ref, wg_ref, bg_ref, idx_ref, wts_ref):
    s = lax.dot_general(
        x_ref[...].astype(jnp.bfloat16),
        wg_ref[...].astype(jnp.bfloat16),
        (((1,), (0,)), ((), ())),
        preferred_element_type=jnp.float32,
    ) + bg_ref[...]
    lanes = lax.broadcasted_iota(jnp.int32, (_GATE_BLK, 128), 1)
    neg = jnp.float32(-1e30)
    s = jnp.where(lanes < E, s, neg)
    m1 = jnp.max(s, axis=1, keepdims=True)
    i1 = jnp.min(jnp.where(s == m1, lanes, 127), axis=1, keepdims=True)
    s2 = jnp.where(lanes == i1, neg, s)
    m2 = jnp.max(s2, axis=1, keepdims=True)
    i2 = jnp.min(jnp.where(s2 == m2, lanes, 127), axis=1, keepdims=True)
    ex = jnp.exp(s - m1)
    z = jnp.sum(ex, axis=1, keepdims=True)
    p1 = 1.0 / z
    p2 = jnp.exp(m2 - m1) / z
    denom = p1 + p2 + jnp.float32(1e-8)
    w1 = p1 / denom
    w2 = p2 / denom
    idx_ref[...] = jnp.where(lanes == 0, i1, jnp.where(lanes == 1, i2, 0))
    wts_ref[...] = jnp.where(lanes == 0, w1, jnp.where(lanes == 1, w2, 0.0))


def _gate(x, wg_t_pad, bg_pad):
    return pl.pallas_call(
        _gate_body,
        grid=(N // _GATE_BLK,),
        in_specs=[
            pl.BlockSpec((_GATE_BLK, D_IN), lambda i: (i, 0)),
            pl.BlockSpec((D_IN, 128), lambda i: (0, 0)),
            pl.BlockSpec((1, 128), lambda i: (0, 0)),
        ],
        out_specs=[
            pl.BlockSpec((_GATE_BLK, 128), lambda i: (i, 0)),
            pl.BlockSpec((_GATE_BLK, 128), lambda i: (i, 0)),
        ],
        out_shape=[
            jax.ShapeDtypeStruct((N, 128), jnp.int32),
            jax.ShapeDtypeStruct((N, 128), jnp.float32),
        ],
    )(x, wg_t_pad, bg_pad)


def _mlp_body(be_ref, bv_ref, xg_ref, w1_ref, b1_ref, w2_ref, b2_ref,
              w3_ref, b3_ref, ws_ref, out_ref, h1t_scr, outt_scr):
    i = pl.program_id(0)
    j = pl.program_id(1)
    valid = bv_ref[i] == 1

    @pl.when(valid & (j == 0))
    def _():
        xt = jnp.transpose(xg_ref[...])
        h1 = lax.dot_general(w1_ref[0], xt, (((1,), (0,)), ((), ())),
                             preferred_element_type=jnp.float32)
        h1t_scr[...] = jnp.maximum(h1 + b1_ref[0], 0.0)

    @pl.when(valid)
    def _():
        h2 = lax.dot_general(w2_ref[0], h1t_scr[...], (((1,), (0,)), ((), ())),
                             preferred_element_type=jnp.float32)
        h2 = jnp.maximum(h2 + b2_ref[0], 0.0)
        part = lax.dot_general(w3_ref[0], h2, (((1,), (0,)), ((), ())),
                               preferred_element_type=jnp.float32)

        @pl.when(j == 0)
        def _():
            outt_scr[...] = part

        @pl.when(j == 1)
        def _():
            acc = outt_scr[...] + part
            out_ref[...] = ((jnp.transpose(acc) + b3_ref[0])
                            * ws_ref[...])


def _mlp(xg, W1, b1c, W2, b2c, W3, b3r, wslot, block_expert, block_valid):
    grid_spec = pltpu.PrefetchScalarGridSpec(
        num_scalar_prefetch=2,
        grid=(NB, 2),
        in_specs=[
            pl.BlockSpec((BLK, D_IN), lambda i, j, be, bv: (i, 0)),
            pl.BlockSpec((1, D_H, D_IN), lambda i, j, be, bv: (be[i], 0, 0)),
            pl.BlockSpec((1, D_H, 1), lambda i, j, be, bv: (be[i], 0, 0)),
            pl.BlockSpec((1, DH2, D_H), lambda i, j, be, bv: (be[i], j, 0)),
            pl.BlockSpec((1, DH2, 1), lambda i, j, be, bv: (be[i], j, 0)),
            pl.BlockSpec((1, D_OUT, DH2), lambda i, j, be, bv: (be[i], 0, j)),
            pl.BlockSpec((1, 1, D_OUT), lambda i, j, be, bv: (be[i], 0, 0)),
            pl.BlockSpec((BLK, 1), lambda i, j, be, bv: (i, 0)),
        ],
        out_specs=pl.BlockSpec((BLK, D_OUT), lambda i, j, be, bv: (i, 0)),
        scratch_shapes=[
            pltpu.VMEM((D_H, BLK), jnp.float32),
            pltpu.VMEM((D_OUT, BLK), jnp.float32),
        ],
    )
    return pl.pallas_call(
        _mlp_body,
        grid_spec=grid_spec,
        out_shape=jax.ShapeDtypeStruct((CAP, D_OUT), jnp.float32),
    )(block_expert, block_valid, xg, W1, b1c, W2, b2c, W3, b3r, wslot)


def _sc_gather(table, ids, nrows, chunk):
    d = table.shape[1]
    rpw = nrows // NW
    nch = rpw // chunk
    mesh = plsc.VectorSubcoreMesh(core_axis_name="c", subcore_axis_name="s")

    @functools.partial(
        pl.kernel,
        mesh=mesh,
        out_type=jax.ShapeDtypeStruct((nrows, d), jnp.float32),
        scratch_types=[
            pltpu.VMEM((chunk,), jnp.int32),
            pltpu.VMEM((chunk, d), jnp.float32),
            pltpu.SemaphoreType.DMA,
        ],
    )
    def k(tab_hbm, ids_hbm, out_hbm, idx_v, rows_v, sem):
        wid = lax.axis_index("s") * 2 + lax.axis_index("c")
        base = wid * rpw

        def body(c, _):
            off = base + c * chunk
            pltpu.sync_copy(ids_hbm.at[pl.ds(off, chunk)], idx_v)
            pltpu.async_copy(tab_hbm.at[idx_v], rows_v, sem).wait()
            pltpu.sync_copy(rows_v, out_hbm.at[pl.ds(off, chunk)])
            return 0

        lax.fori_loop(0, nch, body, 0)

    return k(table, ids)


def _pair_add_body(g_ref, y_ref):
    y_ref[...] = g_ref[:, 0, :] + g_ref[:, 1, :]


def _pair_add(g3):
    return pl.pallas_call(
        _pair_add_body,
        grid=(N // 512,),
        in_specs=[pl.BlockSpec((512, 2, D_OUT), lambda i: (i, 0, 0))],
        out_specs=pl.BlockSpec((512, D_OUT), lambda i: (i, 0)),
        out_shape=jax.ShapeDtypeStruct((N, D_OUT), jnp.float32),
    )(g3)


def kernel(x, W1, b1, W2, b2, W3, b3, Wg, bg):
    f32 = jnp.float32

    b1c = b1.reshape(E, D_H, 1)
    b2c = b2.reshape(E, D_H, 1)
    b3r = b3.reshape(E, 1, D_OUT)
    wg_t_pad = jnp.zeros((D_IN, 128), f32).at[:, :E].set(Wg.T)
    bg_pad = jnp.zeros((1, 128), f32).at[0, :E].set(bg)

    idx2, wts2 = _gate(x, wg_t_pad, bg_pad)
    i1 = idx2[:, 0]
    i2 = idx2[:, 1]
    wv1 = wts2[:, 0]
    wv2 = wts2[:, 1]

    oh = (jax.nn.one_hot(i1, E, dtype=jnp.int32)
          + jax.nn.one_hot(i2, E, dtype=jnp.int32))
    cum = jnp.cumsum(oh, axis=0)
    counts = cum[-1]
    bpe = (counts + BLK - 1) // BLK
    bstart = jnp.concatenate([jnp.zeros((1,), jnp.int32),
                              jnp.cumsum(bpe).astype(jnp.int32)])
    offset = bstart[:E] * BLK
    nbu = bstart[E]

    c1 = jnp.take_along_axis(cum, i1[:, None], axis=1)[:, 0]
    c2 = jnp.take_along_axis(cum, i2[:, None], axis=1)[:, 0]
    pos1 = (offset[i1] + c1 - 1).astype(jnp.int32)
    pos2 = (offset[i2] + c2 - 1).astype(jnp.int32)

    tok = jnp.arange(N, dtype=jnp.int32)
    row_ids = (jnp.zeros((CAP,), jnp.int32)
               .at[pos1].set(tok, unique_indices=True)
               .at[pos2].set(tok, unique_indices=True))
    wslot = (jnp.zeros((CAP,), f32)
             .at[pos1].set(wv1, unique_indices=True)
             .at[pos2].set(wv2, unique_indices=True)).reshape(CAP, 1)

    bids = jnp.arange(NB, dtype=jnp.int32)
    be = jnp.searchsorted(bstart[1:], bids, side="right").astype(jnp.int32)
    be = jnp.minimum(be, E - 1)
    last_e = be[jnp.maximum(nbu - 1, 0)]
    be = jnp.where(bids < nbu, be, last_e)
    bvalid = (bids < nbu).astype(jnp.int32)

    pos_il = jnp.stack([pos1, pos2], axis=1).reshape(2 * N)

    xg = _sc_gather(x, row_ids, CAP, 64)

    outw = _mlp(xg, W1, b1c, W2, b2c, W3, b3r, wslot, be, bvalid)

    g = _sc_gather(outw, pos_il, 2 * N, 64)
    return _pair_add(g.reshape(N, 2, D_OUT))


# speedup vs baseline: 1.2630x; 1.2630x over previous
import functools

import jax
import jax.numpy as jnp
from jax import lax
from jax.experimental import pallas as pl
from jax.experimental.pallas import tpu as pltpu
from jax.experimental.pallas import tpu_sc as plsc

E = 8
K = 2
D_IN = 1024
D_H = 2048
D_OUT = 1024
N = 4096

BLK = 256
NB = 40
CAP = NB * BLK
NW = 32

_GB = 512


def _gate_body(x_ref, wg_ref, bg_ref, idx_ref, wts_ref, cum_ref, xc_ref,
               carry_scr):
    pid = pl.program_id(0)

    @pl.when(pid == 0)
    def _():
        carry_scr[...] = jnp.zeros((1, 128), jnp.float32)

    xb = x_ref[...]
    xc_ref[...] = xb
    s = lax.dot_general(
        xb.astype(jnp.bfloat16), wg_ref[...].astype(jnp.bfloat16),
        (((1,), (0,)), ((), ())),
        preferred_element_type=jnp.float32,
    ) + bg_ref[...]
    lanes = lax.broadcasted_iota(jnp.int32, (_GB, 128), 1)
    neg = jnp.float32(-1e30)
    s = jnp.where(lanes < E, s, neg)
    m1 = jnp.max(s, axis=1, keepdims=True)
    i1 = jnp.min(jnp.where(s == m1, lanes, 127), axis=1, keepdims=True)
    s2 = jnp.where(lanes == i1, neg, s)
    m2 = jnp.max(s2, axis=1, keepdims=True)
    i2 = jnp.min(jnp.where(s2 == m2, lanes, 127), axis=1, keepdims=True)
    ex = jnp.exp(s - m1)
    z = jnp.sum(ex, axis=1, keepdims=True)
    p1 = 1.0 / z
    p2 = jnp.exp(m2 - m1) / z
    denom = p1 + p2 + jnp.float32(1e-8)
    w1 = p1 / denom
    w2 = p2 / denom
    idx_ref[...] = jnp.where(lanes == 0, i1, jnp.where(lanes == 1, i2, 0))
    wts_ref[...] = jnp.where(lanes == 0, w1, jnp.where(lanes == 1, w2, 0.0))

    oh = jnp.where((lanes == i1) | (lanes == i2), 1.0, 0.0)
    r = lax.broadcasted_iota(jnp.int32, (_GB, _GB), 0)
    c = lax.broadcasted_iota(jnp.int32, (_GB, _GB), 1)
    tri = jnp.where(r >= c, 1.0, 0.0).astype(jnp.bfloat16)
    cum = lax.dot_general(tri, oh.astype(jnp.bfloat16),
                          (((1,), (0,)), ((), ())),
                          preferred_element_type=jnp.float32)
    cum = cum + carry_scr[...]
    cum_ref[...] = cum
    carry_scr[...] = cum[_GB - 1:_GB, :]


def _gate(x, wg_t_pad, bg_pad):
    return pl.pallas_call(
        _gate_body,
        grid=(N // _GB,),
        in_specs=[
            pl.BlockSpec((_GB, D_IN), lambda i: (i, 0)),
            pl.BlockSpec((D_IN, 128), lambda i: (0, 0)),
            pl.BlockSpec((1, 128), lambda i: (0, 0)),
        ],
        out_specs=[
            pl.BlockSpec((_GB, 128), lambda i: (i, 0)),
            pl.BlockSpec((_GB, 128), lambda i: (i, 0)),
            pl.BlockSpec((_GB, 128), lambda i: (i, 0)),
            pl.BlockSpec((_GB, D_IN), lambda i: (i, 0)),
        ],
        out_shape=[
            jax.ShapeDtypeStruct((N, 128), jnp.int32),
            jax.ShapeDtypeStruct((N, 128), jnp.float32),
            jax.ShapeDtypeStruct((N, 128), jnp.float32),
            jax.ShapeDtypeStruct((N, D_IN), jnp.float32),
        ],
        scratch_shapes=[pltpu.VMEM((1, 128), jnp.float32)],
    )(x, wg_t_pad, bg_pad)


def _mlp_body(be_ref, bv_ref, xg_ref, w1_ref, b1_ref, w2_ref, b2_ref,
              w3_ref, b3_ref, ws_ref, out_ref):
    i = pl.program_id(0)

    @pl.when(bv_ref[i] == 1)
    def _():
        h1 = lax.dot_general(xg_ref[...], w1_ref[0], (((1,), (1,)), ((), ())),
                             preferred_element_type=jnp.float32)
        h1 = jnp.maximum(h1 + b1_ref[0], 0.0).astype(jnp.bfloat16)
        h2 = lax.dot_general(h1, w2_ref[0], (((1,), (1,)), ((), ())),
                             preferred_element_type=jnp.float32)
        h2 = jnp.maximum(h2 + b2_ref[0], 0.0).astype(jnp.bfloat16)
        o = lax.dot_general(h2, w3_ref[0], (((1,), (1,)), ((), ())),
                            preferred_element_type=jnp.float32)
        out_ref[...] = (o + b3_ref[0]) * ws_ref[...]


def _mlp(xg, W1, b1c, w2b, b2c, w3b, b3r, wslot, block_expert, block_valid):
    grid_spec = pltpu.PrefetchScalarGridSpec(
        num_scalar_prefetch=2,
        grid=(NB,),
        in_specs=[
            pl.BlockSpec((BLK, D_IN), lambda i, be, bv: (i, 0)),
            pl.BlockSpec((1, D_H, D_IN), lambda i, be, bv: (be[i], 0, 0)),
            pl.BlockSpec((1, 1, D_H), lambda i, be, bv: (be[i], 0, 0)),
            pl.BlockSpec((1, D_H, D_H), lambda i, be, bv: (be[i], 0, 0)),
            pl.BlockSpec((1, 1, D_H), lambda i, be, bv: (be[i], 0, 0)),
            pl.BlockSpec((1, D_OUT, D_H), lambda i, be, bv: (be[i], 0, 0)),
            pl.BlockSpec((1, 1, D_OUT), lambda i, be, bv: (be[i], 0, 0)),
            pl.BlockSpec((BLK, 1), lambda i, be, bv: (i, 0)),
        ],
        out_specs=pl.BlockSpec((BLK, D_OUT), lambda i, be, bv: (i, 0)),
    )
    return pl.pallas_call(
        _mlp_body,
        grid_spec=grid_spec,
        out_shape=jax.ShapeDtypeStruct((CAP, D_OUT), jnp.float32),
    )(block_expert, block_valid, xg, W1, b1c, w2b, b2c, w3b, b3r, wslot)


def _sc_gather(table, ids, nrows, chunk):
    d = table.shape[1]
    rpw = nrows // NW
    nch = rpw // chunk
    mesh = plsc.VectorSubcoreMesh(core_axis_name="c", subcore_axis_name="s")

    @functools.partial(
        pl.kernel,
        mesh=mesh,
        out_type=jax.ShapeDtypeStruct((nrows, d), jnp.float32),
        scratch_types=[
            pltpu.VMEM((chunk,), jnp.int32),
            pltpu.VMEM((chunk, d), jnp.float32),
            pltpu.SemaphoreType.DMA,
        ],
    )
    def k(tab_hbm, ids_hbm, out_hbm, idx_v, rows_v, sem):
        wid = lax.axis_index("s") * 2 + lax.axis_index("c")
        base = wid * rpw

        def body(c, _):
            off = base + c * chunk
            pltpu.sync_copy(ids_hbm.at[pl.ds(off, chunk)], idx_v)
            pltpu.async_copy(tab_hbm.at[idx_v], rows_v, sem).wait()
            pltpu.sync_copy(rows_v, out_hbm.at[pl.ds(off, chunk)])
            return 0

        lax.fori_loop(0, nch, body, 0)

    return k(table, ids)


def _pair_add_body(g_ref, y_ref):
    g = g_ref[...].reshape(512, 2, D_OUT)
    y_ref[...] = g[:, 0, :] + g[:, 1, :]


def _pair_add(g):
    return pl.pallas_call(
        _pair_add_body,
        grid=(N // 512,),
        in_specs=[pl.BlockSpec((1024, D_OUT), lambda i: (i, 0))],
        out_specs=pl.BlockSpec((512, D_OUT), lambda i: (i, 0)),
        out_shape=jax.ShapeDtypeStruct((N, D_OUT), jnp.float32),
    )(g)


def kernel(x, W1, b1, W2, b2, W3, b3, Wg, bg):
    f32 = jnp.float32
    i32 = jnp.int32

    w2b = W2.astype(jnp.bfloat16)
    w3b = W3.astype(jnp.bfloat16)
    b1c = b1.reshape(E, 1, D_H)
    b2c = b2.reshape(E, 1, D_H)
    b3r = b3.reshape(E, 1, D_OUT)
    wg_t_pad = jnp.zeros((D_IN, 128), f32).at[:, :E].set(Wg.T)
    bg_pad = jnp.zeros((1, 128), f32).at[0, :E].set(bg)

    idx2, wts2, cumf, xlin = _gate(x, wg_t_pad, bg_pad)
    i1 = idx2[:, 0]
    i2 = idx2[:, 1]
    wv1 = wts2[:, 0]
    wv2 = wts2[:, 1]
    cum = cumf[:, :E].astype(i32)

    counts = cum[-1]
    bpe = (counts + BLK - 1) // BLK
    bstart = jnp.concatenate([jnp.zeros((1,), i32),
                              jnp.cumsum(bpe).astype(i32)])
    offset = bstart[:E] * BLK
    nbu = bstart[E]

    c1 = jnp.take_along_axis(cum, i1[:, None], axis=1)[:, 0]
    c2 = jnp.take_along_axis(cum, i2[:, None], axis=1)[:, 0]
    pos1 = (offset[i1] + c1 - 1).astype(i32)
    pos2 = (offset[i2] + c2 - 1).astype(i32)
    pos_il = jnp.stack([pos1, pos2], axis=1).reshape(2 * N)

    tok = jnp.arange(N, dtype=i32)
    tokf = tok.astype(f32)
    prs = jnp.stack([jnp.stack([tokf, tokf], 1).reshape(2 * N),
                     jnp.stack([wv1, wv2], 1).reshape(2 * N)], axis=1)
    scat = jnp.zeros((CAP, 2), f32).at[pos_il].set(
        prs, unique_indices=True)
    row_ids = scat[:, 0].astype(i32)
    wslot = scat[:, 1:2]

    bids = jnp.arange(NB, dtype=i32)
    be = jnp.sum((bids[:, None] >= bstart[None, 1:]).astype(i32), axis=1)
    be = jnp.minimum(be, E - 1)
    last_e = be[jnp.maximum(nbu - 1, 0)]
    be = jnp.where(bids < nbu, be, last_e)
    bvalid = (bids < nbu).astype(i32)

    xg = _sc_gather(xlin, row_ids, CAP, 64)

    outw = _mlp(xg, W1, b1c, w2b, b2c, w3b, b3r, wslot, be, bvalid)

    g = _sc_gather(outw, pos_il, 2 * N, 64)
    return _pair_add(g)


# speedup vs baseline: 1.6778x; 1.3284x over previous
import functools

import jax
import jax.numpy as jnp
from jax import lax
from jax.experimental import pallas as pl
from jax.experimental.pallas import tpu as pltpu
from jax.experimental.pallas import tpu_sc as plsc

E = 8
K = 2
D_IN = 1024
D_H = 2048
D_OUT = 1024
N = 4096

BLK = 256
NB = 40
CAP = NB * BLK
NW = 32

_GB = 512


def _gate_body(x_ref, wg_ref, bg_ref, idx_ref, wts_ref, cum_ref, xc_ref,
               carry_scr):
    pid = pl.program_id(0)

    @pl.when(pid == 0)
    def _():
        carry_scr[...] = jnp.zeros((1, 128), jnp.float32)

    xb = x_ref[...]
    xc_ref[...] = xb
    s = lax.dot_general(
        xb.astype(jnp.bfloat16), wg_ref[...].astype(jnp.bfloat16),
        (((1,), (0,)), ((), ())),
        preferred_element_type=jnp.float32,
    ) + bg_ref[...]
    lanes = lax.broadcasted_iota(jnp.int32, (_GB, 128), 1)
    neg = jnp.float32(-1e30)
    s = jnp.where(lanes < E, s, neg)
    m1 = jnp.max(s, axis=1, keepdims=True)
    i1 = jnp.min(jnp.where(s == m1, lanes, 127), axis=1, keepdims=True)
    s2 = jnp.where(lanes == i1, neg, s)
    m2 = jnp.max(s2, axis=1, keepdims=True)
    i2 = jnp.min(jnp.where(s2 == m2, lanes, 127), axis=1, keepdims=True)
    ex = jnp.exp(s - m1)
    z = jnp.sum(ex, axis=1, keepdims=True)
    p1 = 1.0 / z
    p2 = jnp.exp(m2 - m1) / z
    denom = p1 + p2 + jnp.float32(1e-8)
    w1 = p1 / denom
    w2 = p2 / denom
    idx_ref[...] = jnp.where(lanes == 0, i1, jnp.where(lanes == 1, i2, 0))
    wts_ref[...] = jnp.where(lanes == 0, w1, jnp.where(lanes == 1, w2, 0.0))

    oh = jnp.where((lanes == i1) | (lanes == i2), 1.0, 0.0)
    r = lax.broadcasted_iota(jnp.int32, (_GB, _GB), 0)
    c = lax.broadcasted_iota(jnp.int32, (_GB, _GB), 1)
    tri = jnp.where(r >= c, 1.0, 0.0).astype(jnp.bfloat16)
    cum = lax.dot_general(tri, oh.astype(jnp.bfloat16),
                          (((1,), (0,)), ((), ())),
                          preferred_element_type=jnp.float32)
    cum = cum + carry_scr[...]
    cum_ref[...] = cum
    carry_scr[...] = cum[_GB - 1:_GB, :]


def _gate(x, wg_t_pad, bg_pad):
    return pl.pallas_call(
        _gate_body,
        grid=(N // _GB,),
        in_specs=[
            pl.BlockSpec((_GB, D_IN), lambda i: (i, 0)),
            pl.BlockSpec((D_IN, 128), lambda i: (0, 0)),
            pl.BlockSpec((1, 128), lambda i: (0, 0)),
        ],
        out_specs=[
            pl.BlockSpec((_GB, 128), lambda i: (i, 0)),
            pl.BlockSpec((_GB, 128), lambda i: (i, 0)),
            pl.BlockSpec((_GB, 128), lambda i: (i, 0)),
            pl.BlockSpec((_GB, D_IN), lambda i: (i, 0)),
        ],
        out_shape=[
            jax.ShapeDtypeStruct((N, 128), jnp.int32),
            jax.ShapeDtypeStruct((N, 128), jnp.float32),
            jax.ShapeDtypeStruct((N, 128), jnp.float32),
            jax.ShapeDtypeStruct((N, D_IN), jnp.float32),
        ],
        scratch_shapes=[pltpu.VMEM((1, 128), jnp.float32)],
    )(x, wg_t_pad, bg_pad)


def _layer_kernel(d_in, d_out, in_dtype, out_dtype, relu):
    def body(be_ref, bv_ref, a_ref, w_ref, b_ref, o_ref):
        i = pl.program_id(0)

        @pl.when(bv_ref[i] == 1)
        def _():
            a = a_ref[...].astype(jnp.float32)
            h = lax.dot_general(a, w_ref[0], (((1,), (1,)), ((), ())),
                                preferred_element_type=jnp.float32)
            h = h + b_ref[0]
            if relu:
                h = jnp.maximum(h, 0.0)
            o_ref[...] = h.astype(out_dtype)

    def run(a, w, b, be, bv):
        grid_spec = pltpu.PrefetchScalarGridSpec(
            num_scalar_prefetch=2,
            grid=(NB,),
            in_specs=[
                pl.BlockSpec((BLK, d_in), lambda i, be, bv: (i, 0)),
                pl.BlockSpec((1, d_out, d_in), lambda i, be, bv: (be[i], 0, 0)),
                pl.BlockSpec((1, 1, d_out), lambda i, be, bv: (be[i], 0, 0)),
            ],
            out_specs=pl.BlockSpec((BLK, d_out), lambda i, be, bv: (i, 0)),
        )
        return pl.pallas_call(
            body,
            grid_spec=grid_spec,
            out_shape=jax.ShapeDtypeStruct((CAP, d_out), out_dtype),
        )(be, bv, a, w, b)

    return run


_layer1 = _layer_kernel(D_IN, D_H, jnp.float32, jnp.bfloat16, True)
_layer2 = _layer_kernel(D_H, D_H, jnp.bfloat16, jnp.bfloat16, True)
_layer3 = _layer_kernel(D_H, D_OUT, jnp.bfloat16, jnp.float32, False)


def _sc_dispatch(xlin, pos1, pos2):
    tpw = N // NW
    ch = 64
    nch = tpw // ch
    mesh = plsc.VectorSubcoreMesh(core_axis_name="c", subcore_axis_name="s")

    @functools.partial(
        pl.kernel,
        mesh=mesh,
        out_type=jax.ShapeDtypeStruct((CAP, D_IN), jnp.float32),
        scratch_types=[
            pltpu.VMEM((ch,), jnp.int32),
            pltpu.VMEM((ch,), jnp.int32),
            pltpu.VMEM((ch, D_IN), jnp.float32),
            pltpu.SemaphoreType.DMA,
            pltpu.SemaphoreType.DMA,
        ],
    )
    def k(x_hbm, p1_hbm, p2_hbm, out_hbm, i1_v, i2_v, rows_v, sem1, sem2):
        wid = lax.axis_index("s") * 2 + lax.axis_index("c")
        base = wid * tpw

        def body(c, _):
            off = base + c * ch
            pltpu.sync_copy(x_hbm.at[pl.ds(off, ch)], rows_v)
            pltpu.sync_copy(p1_hbm.at[pl.ds(off, ch)], i1_v)
            pltpu.sync_copy(p2_hbm.at[pl.ds(off, ch)], i2_v)
            pltpu.async_copy(rows_v, out_hbm.at[i1_v], sem1).wait()
            pltpu.async_copy(rows_v, out_hbm.at[i2_v], sem2).wait()
            return 0

        lax.fori_loop(0, nch, body, 0)

    return k(xlin, pos1, pos2)


def _sc_pair_gather(outw, pos_il):
    rpw = 2 * N // NW
    ch = 64
    nch = rpw // ch
    mesh = plsc.VectorSubcoreMesh(core_axis_name="c", subcore_axis_name="s")

    @functools.partial(
        pl.kernel,
        mesh=mesh,
        out_type=jax.ShapeDtypeStruct((2 * N, D_OUT), jnp.float32),
        scratch_types=[
            pltpu.VMEM((ch,), jnp.int32),
            pltpu.VMEM((ch, D_OUT), jnp.float32),
            pltpu.SemaphoreType.DMA,
        ],
    )
    def k(tab_hbm, ids_hbm, out_hbm, idx_v, rows_v, sem):
        wid = lax.axis_index("s") * 2 + lax.axis_index("c")
        base = wid * rpw

        def body(c, _):
            off = base + c * ch
            pltpu.sync_copy(ids_hbm.at[pl.ds(off, ch)], idx_v)
            pltpu.async_copy(tab_hbm.at[idx_v], rows_v, sem).wait()
            pltpu.sync_copy(rows_v, out_hbm.at[pl.ds(off, ch)])
            return 0

        lax.fori_loop(0, nch, body, 0)

    return k(outw, pos_il)


def _pair_add_body(g_ref, w_ref, y_ref):
    g = g_ref[...].reshape(512, 2, D_OUT)
    w1 = w_ref[:, 0:1]
    w2 = w_ref[:, 1:2]
    y_ref[...] = g[:, 0, :] * w1 + g[:, 1, :] * w2


def _pair_add(g, wts2):
    return pl.pallas_call(
        _pair_add_body,
        grid=(N // 512,),
        in_specs=[
            pl.BlockSpec((1024, D_OUT), lambda i: (i, 0)),
            pl.BlockSpec((512, 128), lambda i: (i, 0)),
        ],
        out_specs=pl.BlockSpec((512, D_OUT), lambda i: (i, 0)),
        out_shape=jax.ShapeDtypeStruct((N, D_OUT), jnp.float32),
    )(g, wts2)


def kernel(x, W1, b1, W2, b2, W3, b3, Wg, bg):
    f32 = jnp.float32
    i32 = jnp.int32

    b1c = b1.reshape(E, 1, D_H)
    b2c = b2.reshape(E, 1, D_H)
    b3r = b3.reshape(E, 1, D_OUT)
    wg_t_pad = jnp.zeros((D_IN, 128), f32).at[:, :E].set(Wg.T)
    bg_pad = jnp.zeros((1, 128), f32).at[0, :E].set(bg)

    idx2, wts2, cumf, xlin = _gate(x, wg_t_pad, bg_pad)
    i1 = idx2[:, 0]
    i2 = idx2[:, 1]
    cum = cumf[:, :E].astype(i32)

    counts = cum[-1]
    bpe = (counts + BLK - 1) // BLK
    bstart = jnp.concatenate([jnp.zeros((1,), i32),
                              jnp.cumsum(bpe).astype(i32)])
    offset = bstart[:E] * BLK
    nbu = bstart[E]

    c1 = jnp.take_along_axis(cum, i1[:, None], axis=1)[:, 0]
    c2 = jnp.take_along_axis(cum, i2[:, None], axis=1)[:, 0]
    pos1 = (offset[i1] + c1 - 1).astype(i32)
    pos2 = (offset[i2] + c2 - 1).astype(i32)
    pos_il = jnp.stack([pos1, pos2], axis=1).reshape(2 * N)

    bids = jnp.arange(NB, dtype=i32)
    be = jnp.sum((bids[:, None] >= bstart[None, 1:]).astype(i32), axis=1)
    be = jnp.minimum(be, E - 1)
    last_e = be[jnp.maximum(nbu - 1, 0)]
    be = jnp.where(bids < nbu, be, last_e)
    bvalid = (bids < nbu).astype(i32)

    xg = _sc_dispatch(xlin, pos1, pos2)

    h1 = _layer1(xg, W1, b1c, be, bvalid)
    h2 = _layer2(h1, W2, b2c, be, bvalid)
    outw = _layer3(h2, W3, b3r, be, bvalid)

    g = _sc_pair_gather(outw, pos_il)
    return _pair_add(g, wts2)


# speedup vs baseline: 1.7101x; 1.0192x over previous
import functools

import jax
import jax.numpy as jnp
from jax import lax
from jax.experimental import pallas as pl
from jax.experimental.pallas import tpu as pltpu
from jax.experimental.pallas import tpu_sc as plsc

E = 8
K = 2
D_IN = 1024
D_H = 2048
D_OUT = 1024
N = 4096

BLK = 512
NB = 24
CAP = NB * BLK
NW = 32

_GB = 512


def _gate_body(x_ref, wg_ref, bg_ref, idx_ref, wts_ref, cum_ref, xc_ref,
               carry_scr):
    pid = pl.program_id(0)

    @pl.when(pid == 0)
    def _():
        carry_scr[...] = jnp.zeros((1, 128), jnp.float32)

    xb = x_ref[...]
    xc_ref[...] = xb
    s = lax.dot_general(
        xb.astype(jnp.bfloat16), wg_ref[...].astype(jnp.bfloat16),
        (((1,), (0,)), ((), ())),
        preferred_element_type=jnp.float32,
    ) + bg_ref[...]
    lanes = lax.broadcasted_iota(jnp.int32, (_GB, 128), 1)
    neg = jnp.float32(-1e30)
    s = jnp.where(lanes < E, s, neg)
    m1 = jnp.max(s, axis=1, keepdims=True)
    i1 = jnp.min(jnp.where(s == m1, lanes, 127), axis=1, keepdims=True)
    s2 = jnp.where(lanes == i1, neg, s)
    m2 = jnp.max(s2, axis=1, keepdims=True)
    i2 = jnp.min(jnp.where(s2 == m2, lanes, 127), axis=1, keepdims=True)
    ex = jnp.exp(s - m1)
    z = jnp.sum(ex, axis=1, keepdims=True)
    p1 = 1.0 / z
    p2 = jnp.exp(m2 - m1) / z
    denom = p1 + p2 + jnp.float32(1e-8)
    w1 = p1 / denom
    w2 = p2 / denom
    idx_ref[...] = jnp.where(lanes == 0, i1, jnp.where(lanes == 1, i2, 0))
    wts_ref[...] = jnp.where(lanes == 0, w1, jnp.where(lanes == 1, w2, 0.0))

    oh = jnp.where((lanes == i1) | (lanes == i2), 1.0, 0.0)
    r = lax.broadcasted_iota(jnp.int32, (_GB, _GB), 0)
    c = lax.broadcasted_iota(jnp.int32, (_GB, _GB), 1)
    tri = jnp.where(r >= c, 1.0, 0.0).astype(jnp.bfloat16)
    cum = lax.dot_general(tri, oh.astype(jnp.bfloat16),
                          (((1,), (0,)), ((), ())),
                          preferred_element_type=jnp.float32)
    cum = cum + carry_scr[...]
    cum_ref[...] = cum
    carry_scr[...] = cum[_GB - 1:_GB, :]


def _gate(x, wg_t_pad, bg_pad):
    return pl.pallas_call(
        _gate_body,
        grid=(N // _GB,),
        in_specs=[
            pl.BlockSpec((_GB, D_IN), lambda i: (i, 0)),
            pl.BlockSpec((D_IN, 128), lambda i: (0, 0)),
            pl.BlockSpec((1, 128), lambda i: (0, 0)),
        ],
        out_specs=[
            pl.BlockSpec((_GB, 128), lambda i: (i, 0)),
            pl.BlockSpec((_GB, 128), lambda i: (i, 0)),
            pl.BlockSpec((_GB, 128), lambda i: (i, 0)),
            pl.BlockSpec((_GB, D_IN), lambda i: (i, 0)),
        ],
        out_shape=[
            jax.ShapeDtypeStruct((N, 128), jnp.int32),
            jax.ShapeDtypeStruct((N, 128), jnp.float32),
            jax.ShapeDtypeStruct((N, 128), jnp.float32),
            jax.ShapeDtypeStruct((N, D_IN), jnp.float32),
        ],
        scratch_shapes=[pltpu.VMEM((1, 128), jnp.float32)],
    )(x, wg_t_pad, bg_pad)


def _layer_kernel(d_in, d_out, in_dtype, out_dtype, relu):
    def body(be_ref, bv_ref, a_ref, w_ref, b_ref, o_ref):
        i = pl.program_id(0)

        @pl.when(bv_ref[i] == 1)
        def _():
            a = a_ref[...].astype(jnp.float32)
            h = lax.dot_general(a, w_ref[0], (((1,), (1,)), ((), ())),
                                preferred_element_type=jnp.float32)
            h = h + b_ref[0]
            if relu:
                h = jnp.maximum(h, 0.0)
            o_ref[...] = h.astype(out_dtype)

    def run(a, w, b, be, bv):
        grid_spec = pltpu.PrefetchScalarGridSpec(
            num_scalar_prefetch=2,
            grid=(NB,),
            in_specs=[
                pl.BlockSpec((BLK, d_in), lambda i, be, bv: (i, 0)),
                pl.BlockSpec((1, d_out, d_in), lambda i, be, bv: (be[i], 0, 0)),
                pl.BlockSpec((1, 1, d_out), lambda i, be, bv: (be[i], 0, 0)),
            ],
            out_specs=pl.BlockSpec((BLK, d_out), lambda i, be, bv: (i, 0)),
        )
        return pl.pallas_call(
            body,
            grid_spec=grid_spec,
            out_shape=jax.ShapeDtypeStruct((CAP, d_out), out_dtype),
        )(be, bv, a, w, b)

    return run


_layer1 = _layer_kernel(D_IN, D_H, jnp.float32, jnp.bfloat16, True)
_layer2 = _layer_kernel(D_H, D_H, jnp.bfloat16, jnp.bfloat16, True)
_layer3 = _layer_kernel(D_H, D_OUT, jnp.bfloat16, jnp.float32, False)


def _sc_dispatch(xlin, pos1, pos2):
    tpw = N // NW
    ch = 64
    nch = tpw // ch
    mesh = plsc.VectorSubcoreMesh(core_axis_name="c", subcore_axis_name="s")

    @functools.partial(
        pl.kernel,
        mesh=mesh,
        out_type=jax.ShapeDtypeStruct((CAP, D_IN), jnp.float32),
        scratch_types=[
            pltpu.VMEM((ch,), jnp.int32),
            pltpu.VMEM((ch,), jnp.int32),
            pltpu.VMEM((ch, D_IN), jnp.float32),
            pltpu.SemaphoreType.DMA,
            pltpu.SemaphoreType.DMA,
        ],
    )
    def k(x_hbm, p1_hbm, p2_hbm, out_hbm, i1_v, i2_v, rows_v, sem1, sem2):
        wid = lax.axis_index("s") * 2 + lax.axis_index("c")
        base = wid * tpw

        def body(c, _):
            off = base + c * ch
            pltpu.sync_copy(x_hbm.at[pl.ds(off, ch)], rows_v)
            pltpu.sync_copy(p1_hbm.at[pl.ds(off, ch)], i1_v)
            pltpu.sync_copy(p2_hbm.at[pl.ds(off, ch)], i2_v)
            pltpu.async_copy(rows_v, out_hbm.at[i1_v], sem1).wait()
            pltpu.async_copy(rows_v, out_hbm.at[i2_v], sem2).wait()
            return 0

        lax.fori_loop(0, nch, body, 0)

    return k(xlin, pos1, pos2)


def _sc_pair_gather(outw, pos_il):
    rpw = 2 * N // NW
    ch = 64
    nch = rpw // ch
    mesh = plsc.VectorSubcoreMesh(core_axis_name="c", subcore_axis_name="s")

    @functools.partial(
        pl.kernel,
        mesh=mesh,
        out_type=jax.ShapeDtypeStruct((2 * N, D_OUT), jnp.float32),
        scratch_types=[
            pltpu.VMEM((ch,), jnp.int32),
            pltpu.VMEM((ch, D_OUT), jnp.float32),
            pltpu.SemaphoreType.DMA,
        ],
    )
    def k(tab_hbm, ids_hbm, out_hbm, idx_v, rows_v, sem):
        wid = lax.axis_index("s") * 2 + lax.axis_index("c")
        base = wid * rpw

        def body(c, _):
            off = base + c * ch
            pltpu.sync_copy(ids_hbm.at[pl.ds(off, ch)], idx_v)
            pltpu.async_copy(tab_hbm.at[idx_v], rows_v, sem).wait()
            pltpu.sync_copy(rows_v, out_hbm.at[pl.ds(off, ch)])
            return 0

        lax.fori_loop(0, nch, body, 0)

    return k(outw, pos_il)


def _pair_add_body(g_ref, w_ref, y_ref):
    g = g_ref[...].reshape(512, 2, D_OUT)
    w1 = w_ref[:, 0:1]
    w2 = w_ref[:, 1:2]
    y_ref[...] = g[:, 0, :] * w1 + g[:, 1, :] * w2


def _pair_add(g, wts2):
    return pl.pallas_call(
        _pair_add_body,
        grid=(N // 512,),
        in_specs=[
            pl.BlockSpec((1024, D_OUT), lambda i: (i, 0)),
            pl.BlockSpec((512, 128), lambda i: (i, 0)),
        ],
        out_specs=pl.BlockSpec((512, D_OUT), lambda i: (i, 0)),
        out_shape=jax.ShapeDtypeStruct((N, D_OUT), jnp.float32),
    )(g, wts2)


def kernel(x, W1, b1, W2, b2, W3, b3, Wg, bg):
    f32 = jnp.float32
    i32 = jnp.int32

    b1c = b1.reshape(E, 1, D_H)
    b2c = b2.reshape(E, 1, D_H)
    b3r = b3.reshape(E, 1, D_OUT)
    wg_t_pad = jnp.zeros((D_IN, 128), f32).at[:, :E].set(Wg.T)
    bg_pad = jnp.zeros((1, 128), f32).at[0, :E].set(bg)

    idx2, wts2, cumf, xlin = _gate(x, wg_t_pad, bg_pad)
    i1 = idx2[:, 0]
    i2 = idx2[:, 1]
    cum = cumf[:, :E].astype(i32)

    counts = cum[-1]
    bpe = (counts + BLK - 1) // BLK
    bstart = jnp.concatenate([jnp.zeros((1,), i32),
                              jnp.cumsum(bpe).astype(i32)])
    offset = bstart[:E] * BLK
    nbu = bstart[E]

    c1 = jnp.take_along_axis(cum, i1[:, None], axis=1)[:, 0]
    c2 = jnp.take_along_axis(cum, i2[:, None], axis=1)[:, 0]
    pos1 = (offset[i1] + c1 - 1).astype(i32)
    pos2 = (offset[i2] + c2 - 1).astype(i32)
    pos_il = jnp.stack([pos1, pos2], axis=1).reshape(2 * N)

    bids = jnp.arange(NB, dtype=i32)
    be = jnp.sum((bids[:, None] >= bstart[None, 1:]).astype(i32), axis=1)
    be = jnp.minimum(be, E - 1)
    last_e = be[jnp.maximum(nbu - 1, 0)]
    be = jnp.where(bids < nbu, be, last_e)
    bvalid = (bids < nbu).astype(i32)

    xg = _sc_dispatch(xlin, pos1, pos2)

    h1 = _layer1(xg, W1, b1c, be, bvalid)
    h2 = _layer2(h1, W2, b2c, be, bvalid)
    outw = _layer3(h2, W3, b3r, be, bvalid)

    g = _sc_pair_gather(outw, pos_il)
    return _pair_add(g, wts2)
